# baseline (speedup 1.0000x reference)
import jax
import jax.numpy as jnp
from jax.experimental import pallas as pl

N_PARTS = 2
N_EMBED = 8192
CODE_DIM = 32
COMMITMENT = 1.0

_TM = 256


def _vq_block(x_ref, cb_ref, q_ref, idx_ref, commit_ref):
    x = x_ref[...]
    cb = cb_ref[...]
    xsq = jnp.sum(x ** 2, axis=-1, keepdims=True)
    csq = jnp.sum(cb ** 2, axis=-1)[None, :]
    dist = xsq - 2.0 * jnp.dot(x, cb.T, preferred_element_type=jnp.float32) + csq
    idx = jnp.argmin(dist, axis=-1).astype(jnp.int32)
    onehot = (jax.lax.broadcasted_iota(jnp.int32, dist.shape, 1)
              == idx[:, None]).astype(jnp.float32)
    q = jnp.dot(onehot, cb, preferred_element_type=jnp.float32)
    q_ref[...] = q
    idx_ref[0, 0, :] = idx
    commit_ref[0, 0, :] = jnp.full((8,), jnp.sum((q - x) ** 2), jnp.float32)


def kernel(z_e, codebook):
    B, H, W, D = z_e.shape
    d = D // N_PARTS
    n_tok = B * H * W
    total = n_tok * N_PARTS
    grid = total // _TM

    x_all = z_e.reshape(n_tok, N_PARTS, d).transpose(1, 0, 2).reshape(total, d)

    q_all, idx3, commit_parts = pl.pallas_call(
        _vq_block,
        grid=(grid,),
        in_specs=[
            pl.BlockSpec((_TM, d), lambda i: (i, 0)),
            pl.BlockSpec((N_EMBED, d), lambda i: (0, 0)),
        ],
        out_specs=[
            pl.BlockSpec((_TM, d), lambda i: (i, 0)),
            pl.BlockSpec((1, 1, _TM), lambda i: (i, 0, 0)),
            pl.BlockSpec((1, 1, 8), lambda i: (i, 0, 0)),
        ],
        out_shape=[
            jax.ShapeDtypeStruct((total, d), jnp.float32),
            jax.ShapeDtypeStruct((grid, 1, _TM), jnp.int32),
            jax.ShapeDtypeStruct((grid, 1, 8), jnp.float32),
        ],
    )(x_all, codebook)

    z_q = (q_all.reshape(N_PARTS, B, H, W, d)
           .transpose(1, 2, 3, 0, 4).reshape(B, H, W, D))
    indices = (idx3.reshape(N_PARTS, n_tok)
               .reshape(N_PARTS, B, H * W).transpose(1, 2, 0))
    commit_loss = (COMMITMENT * jnp.sum(commit_parts[:, 0, 0])
                   / jnp.float32(n_tok * d))
    return z_q, indices, commit_loss


# speedup vs baseline: 1.0863x; 1.0863x over previous
import jax
import jax.numpy as jnp
from jax.experimental import pallas as pl
from jax.experimental.pallas import tpu as pltpu

N_PARTS = 2
N_EMBED = 8192
CODE_DIM = 32
COMMITMENT = 1.0

_TM = 256


def _vq_block(x_ref, cb_ref, q_ref, idx_ref, commit_ref, cbt_ref, csq_ref):
    i = pl.program_id(0)

    @pl.when(i == 0)
    def _build_aug():
        cb0 = cb_ref[...]
        cbt_ref[...] = (-2.0 * cb0).T
        csq_ref[...] = jnp.sum(cb0 * cb0, axis=-1)[None, :]

    x = x_ref[...]
    scores = jnp.dot(x, cbt_ref[...],
                     preferred_element_type=jnp.float32) + csq_ref[...]
    idx = jnp.argmin(scores, axis=-1).astype(jnp.int32)
    onehot = (jax.lax.broadcasted_iota(jnp.int32, scores.shape, 1)
              == idx[:, None]).astype(jnp.float32)
    q = jnp.dot(onehot, cb_ref[...], preferred_element_type=jnp.float32)
    q_ref[...] = q
    idx_ref[0, 0, :] = idx
    commit_ref[0, 0, :] = jnp.full((8,), jnp.sum((q - x) ** 2), jnp.float32)


def kernel(z_e, codebook):
    B, H, W, D = z_e.shape
    d = D // N_PARTS
    n_tok = B * H * W
    total = n_tok * N_PARTS
    grid = total // _TM

    x_all = z_e.reshape(n_tok, N_PARTS, d).transpose(1, 0, 2).reshape(total, d)

    q_all, idx3, commit_parts = pl.pallas_call(
        _vq_block,
        grid=(grid,),
        in_specs=[
            pl.BlockSpec((_TM, d), lambda i: (i, 0)),
            pl.BlockSpec((N_EMBED, d), lambda i: (0, 0)),
        ],
        out_specs=[
            pl.BlockSpec((_TM, d), lambda i: (i, 0)),
            pl.BlockSpec((1, 1, _TM), lambda i: (i, 0, 0)),
            pl.BlockSpec((1, 1, 8), lambda i: (i, 0, 0)),
        ],
        out_shape=[
            jax.ShapeDtypeStruct((total, d), jnp.float32),
            jax.ShapeDtypeStruct((grid, 1, _TM), jnp.int32),
            jax.ShapeDtypeStruct((grid, 1, 8), jnp.float32),
        ],
        scratch_shapes=[
            pltpu.VMEM((d, N_EMBED), jnp.float32),
            pltpu.VMEM((1, N_EMBED), jnp.float32),
        ],
    )(x_all, codebook)

    z_q = (q_all.reshape(N_PARTS, B, H, W, d)
           .transpose(1, 2, 3, 0, 4).reshape(B, H, W, D))
    indices = (idx3.reshape(N_PARTS, n_tok)
               .reshape(N_PARTS, B, H * W).transpose(1, 2, 0))
    commit_loss = (COMMITMENT * jnp.sum(commit_parts[:, 0, 0])
                   / jnp.float32(n_tok * d))
    return z_q, indices, commit_loss


# speedup vs baseline: 1.1455x; 1.0546x over previous
import jax
import jax.numpy as jnp
from jax.experimental import pallas as pl
from jax.experimental.pallas import tpu as pltpu

N_PARTS = 2
N_EMBED = 8192
CODE_DIM = 32
COMMITMENT = 1.0

_TM = 512


def _vq_block(x_ref, cb_ref, q_ref, idx_ref, commit_ref, cbt_ref, csq_ref):
    i = pl.program_id(0)

    @pl.when(i == 0)
    def _build_aug():
        cb0 = cb_ref[...]
        cbt_ref[...] = (-2.0 * cb0).T
        csq_ref[...] = jnp.sum(cb0 * cb0, axis=-1)[None, :]

    x = x_ref[...]
    scores = jnp.dot(x, cbt_ref[...],
                     preferred_element_type=jnp.float32) + csq_ref[...]
    idx = jnp.argmin(scores, axis=-1).astype(jnp.int32)
    onehot = (jax.lax.broadcasted_iota(jnp.int32, scores.shape, 1)
              == idx[:, None]).astype(jnp.float32)
    q = jnp.dot(onehot, cb_ref[...], preferred_element_type=jnp.float32)
    q_ref[...] = q
    idx_ref[0, 0, :] = idx
    commit_ref[0, 0, :] = jnp.full((8,), jnp.sum((q - x) ** 2), jnp.float32)


def kernel(z_e, codebook):
    B, H, W, D = z_e.shape
    d = D // N_PARTS
    n_tok = B * H * W
    total = n_tok * N_PARTS
    grid = total // _TM

    x_all = z_e.reshape(n_tok, N_PARTS, d).transpose(1, 0, 2).reshape(total, d)

    q_all, idx3, commit_parts = pl.pallas_call(
        _vq_block,
        grid=(grid,),
        in_specs=[
            pl.BlockSpec((_TM, d), lambda i: (i, 0)),
            pl.BlockSpec((N_EMBED, d), lambda i: (0, 0)),
        ],
        out_specs=[
            pl.BlockSpec((_TM, d), lambda i: (i, 0)),
            pl.BlockSpec((1, 1, _TM), lambda i: (i, 0, 0)),
            pl.BlockSpec((1, 1, 8), lambda i: (i, 0, 0)),
        ],
        out_shape=[
            jax.ShapeDtypeStruct((total, d), jnp.float32),
            jax.ShapeDtypeStruct((grid, 1, _TM), jnp.int32),
            jax.ShapeDtypeStruct((grid, 1, 8), jnp.float32),
        ],
        scratch_shapes=[
            pltpu.VMEM((d, N_EMBED), jnp.float32),
            pltpu.VMEM((1, N_EMBED), jnp.float32),
        ],
    )(x_all, codebook)

    z_q = (q_all.reshape(N_PARTS, B, H, W, d)
           .transpose(1, 2, 3, 0, 4).reshape(B, H, W, D))
    indices = (idx3.reshape(N_PARTS, n_tok)
               .reshape(N_PARTS, B, H * W).transpose(1, 2, 0))
    commit_loss = (COMMITMENT * jnp.sum(commit_parts[:, 0, 0])
                   / jnp.float32(n_tok * d))
    return z_q, indices, commit_loss


# speedup vs baseline: 1.6420x; 1.4333x over previous
import jax
import jax.numpy as jnp
from jax.experimental import pallas as pl
from jax.experimental.pallas import tpu as pltpu

N_PARTS = 2
N_EMBED = 8192
CODE_DIM = 32
COMMITMENT = 1.0

_TM = 256


def _vq_block(x_ref, cb_ref, q_ref, idx_ref, commit_ref, cbt_ref, csq_ref):
    i = pl.program_id(0)

    @pl.when(i == 0)
    def _build_aug():
        cb0 = cb_ref[...]
        cbt_ref[...] = (-2.0 * cb0).T
        csq_ref[...] = jnp.sum(cb0 * cb0, axis=-1)[None, :]

    x64 = x_ref[...]
    csq = csq_ref[...]
    idx_parts = []
    commit = jnp.float32(0.0)
    for p in range(N_PARTS):
        x = x64[:, p * CODE_DIM:(p + 1) * CODE_DIM]
        scores = jnp.dot(x, cbt_ref[...],
                         preferred_element_type=jnp.float32) + csq
        idx = jnp.argmin(scores, axis=-1).astype(jnp.int32)
        onehot = (jax.lax.broadcasted_iota(jnp.int32, scores.shape, 1)
                  == idx[:, None]).astype(jnp.float32)
        q = jnp.dot(onehot, cb_ref[...], preferred_element_type=jnp.float32)
        q_ref[:, p * CODE_DIM:(p + 1) * CODE_DIM] = q
        idx_parts.append(idx[:, None])
        commit = commit + jnp.sum((q - x) ** 2)
    idx_ref[0, :, :] = jnp.concatenate(idx_parts, axis=1)
    commit_ref[0, 0, :] = jnp.full((8,), commit, jnp.float32)


def kernel(z_e, codebook):
    B, H, W, D = z_e.shape
    d = D // N_PARTS
    n_tok = B * H * W
    n_blk = n_tok // _TM

    x_v = z_e.reshape(n_tok, D)

    q_all, idx3, commit_parts = pl.pallas_call(
        _vq_block,
        grid=(n_blk,),
        in_specs=[
            pl.BlockSpec((_TM, D), lambda i: (i, 0)),
            pl.BlockSpec((N_EMBED, d), lambda i: (0, 0)),
        ],
        out_specs=[
            pl.BlockSpec((_TM, D), lambda i: (i, 0)),
            pl.BlockSpec((1, _TM, N_PARTS), lambda i: (i, 0, 0)),
            pl.BlockSpec((1, 1, 8), lambda i: (i, 0, 0)),
        ],
        out_shape=[
            jax.ShapeDtypeStruct((n_tok, D), jnp.float32),
            jax.ShapeDtypeStruct((n_blk, _TM, N_PARTS), jnp.int32),
            jax.ShapeDtypeStruct((n_blk, 1, 8), jnp.float32),
        ],
        scratch_shapes=[
            pltpu.VMEM((d, N_EMBED), jnp.float32),
            pltpu.VMEM((1, N_EMBED), jnp.float32),
        ],
    )(x_v, codebook)

    z_q = q_all.reshape(B, H, W, D)
    indices = idx3.reshape(B, H * W, N_PARTS)
    commit_loss = (COMMITMENT * jnp.sum(commit_parts[:, 0, 0])
                   / jnp.float32(n_tok * d))
    return z_q, indices, commit_loss


# speedup vs baseline: 3.1275x; 1.9047x over previous
import functools

import jax
import jax.numpy as jnp
from jax import lax
from jax.experimental import pallas as pl
from jax.experimental.pallas import tpu as pltpu
from jax.experimental.pallas import tpu_sc as plsc

N_PARTS = 2
N_EMBED = 8192
CODE_DIM = 32
COMMITMENT = 1.0

_TM = 256
_NC, _NS = 2, 16
_NW = _NC * _NS
_CHUNK = 128


def _vq_block(x_ref, cb_ref, idx_ref, cbt_ref, csq_ref):
    i = pl.program_id(0)

    @pl.when(i == 0)
    def _build_aug():
        cb0 = cb_ref[...]
        cbt_ref[...] = (-2.0 * cb0).T
        csq_ref[...] = jnp.sum(cb0 * cb0, axis=-1)[None, :]

    x64 = x_ref[...]
    csq = csq_ref[...]
    idx_parts = []
    for p in range(N_PARTS):
        x = x64[:, p * CODE_DIM:(p + 1) * CODE_DIM]
        scores = jnp.dot(x, cbt_ref[...],
                         preferred_element_type=jnp.float32) + csq
        idx = jnp.argmin(scores, axis=-1).astype(jnp.int32)
        idx_parts.append(idx[:, None])
    idx_ref[0, :, :] = jnp.concatenate(idx_parts, axis=1)


def _sc_gather_commit(cb_hbm, idx_hbm, x_hbm, q_hbm, part_hbm,
                      idx_v, rows_v, x_v, acc_v, sem):
    wid = lax.axis_index("s") * _NC + lax.axis_index("c")
    nchunk = idx_v.shape[0]
    base = wid * nchunk
    pltpu.sync_copy(idx_hbm.at[pl.ds(base, nchunk)], idx_v)
    copies = [
        pltpu.async_copy(cb_hbm.at[idx_v.at[j]], rows_v.at[j], sem)
        for j in range(nchunk)
    ]
    pltpu.sync_copy(x_hbm.at[pl.ds(base, nchunk)], x_v)
    for cp in copies:
        cp.wait()
    for j in range(nchunk):
        pltpu.sync_copy(rows_v.at[j], q_hbm.at[base + j])
    acc_v[...] = jnp.zeros((16,), jnp.float32)

    def _row(r, j):
        a0 = rows_v[j, r, pl.ds(0, 16)] - x_v[j, r, pl.ds(0, 16)]
        a1 = rows_v[j, r, pl.ds(16, 16)] - x_v[j, r, pl.ds(16, 16)]
        acc_v[...] = acc_v[...] + a0 * a0 + a1 * a1
        return j

    for j in range(nchunk):
        lax.fori_loop(0, _CHUNK, _row, j)
    pltpu.sync_copy(acc_v, part_hbm.at[wid])


def kernel(z_e, codebook):
    B, H, W, D = z_e.shape
    d = D // N_PARTS
    n_tok = B * H * W
    n_blk = n_tok // _TM
    total = n_tok * N_PARTS
    nchunk = total // (_NW * _CHUNK)

    x_v = z_e.reshape(n_tok, D)

    idx3 = pl.pallas_call(
        _vq_block,
        grid=(n_blk,),
        in_specs=[
            pl.BlockSpec((_TM, D), lambda i: (i, 0)),
            pl.BlockSpec((N_EMBED, d), lambda i: (0, 0)),
        ],
        out_specs=pl.BlockSpec((1, _TM, N_PARTS), lambda i: (i, 0, 0)),
        out_shape=jax.ShapeDtypeStruct((n_blk, _TM, N_PARTS), jnp.int32),
        scratch_shapes=[
            pltpu.VMEM((d, N_EMBED), jnp.float32),
            pltpu.VMEM((1, N_EMBED), jnp.float32),
        ],
    )(x_v, codebook)

    idx_chunks = idx3.reshape(total // _CHUNK, _CHUNK)
    x_rows = z_e.reshape(total // _CHUNK, _CHUNK, d)

    sc = functools.partial(
        pl.kernel,
        mesh=plsc.VectorSubcoreMesh(core_axis_name="c", subcore_axis_name="s"),
        compiler_params=pltpu.CompilerParams(use_tc_tiling_on_sc=False),
        out_type=[
            jax.ShapeDtypeStruct((total // _CHUNK, _CHUNK, d), jnp.float32),
            jax.ShapeDtypeStruct((_NW, 16), jnp.float32),
        ],
        scratch_types=[
            pltpu.VMEM((nchunk, _CHUNK), jnp.int32),
            pltpu.VMEM((nchunk, _CHUNK, d), jnp.float32),
            pltpu.VMEM((nchunk, _CHUNK, d), jnp.float32),
            pltpu.VMEM((16,), jnp.float32),
            pltpu.SemaphoreType.DMA,
        ],
    )(_sc_gather_commit)
    q_rows, partials = sc(codebook, idx_chunks, x_rows)

    z_q = q_rows.reshape(B, H, W, D)
    indices = idx3.reshape(B, H * W, N_PARTS)
    commit_loss = (COMMITMENT * jnp.sum(partials)
                   / jnp.float32(n_tok * d))
    return z_q, indices, commit_loss


# speedup vs baseline: 3.1947x; 1.0215x over previous
import functools

import jax
import jax.numpy as jnp
from jax import lax
from jax.experimental import pallas as pl
from jax.experimental.pallas import tpu as pltpu
from jax.experimental.pallas import tpu_sc as plsc

N_PARTS = 2
N_EMBED = 8192
CODE_DIM = 32
COMMITMENT = 1.0

_TM = 512
_NC, _NS = 2, 16
_NW = _NC * _NS
_CHUNK = 128


def _vq_block(x_ref, cb_ref, idx_ref, cbt_ref, csq_ref):
    i = pl.program_id(0)

    @pl.when(i == 0)
    def _build_aug():
        cb0 = cb_ref[...]
        cbt_ref[...] = (-2.0 * cb0).T
        csq_ref[...] = jnp.sum(cb0 * cb0, axis=-1)[None, :]

    x64 = x_ref[...]
    csq = csq_ref[...]
    idx_parts = []
    for p in range(N_PARTS):
        x = x64[:, p * CODE_DIM:(p + 1) * CODE_DIM]
        scores = jnp.dot(x, cbt_ref[...],
                         preferred_element_type=jnp.float32) + csq
        idx = jnp.argmin(scores, axis=-1).astype(jnp.int32)
        idx_parts.append(idx[:, None])
    idx_ref[0, :, :] = jnp.concatenate(idx_parts, axis=1)


def _sc_gather_commit(cb_hbm, idx_hbm, x_hbm, q_hbm, part_hbm,
                      idx_v, rows_v, x_v, acc_v, sem):
    wid = lax.axis_index("s") * _NC + lax.axis_index("c")
    nchunk = idx_v.shape[0]
    base = wid * nchunk
    pltpu.sync_copy(idx_hbm.at[pl.ds(base, nchunk)], idx_v)
    copies = [
        pltpu.async_copy(cb_hbm.at[idx_v.at[j]], rows_v.at[j], sem)
        for j in range(nchunk)
    ]
    pltpu.sync_copy(x_hbm.at[pl.ds(base, nchunk)], x_v)
    for cp in copies:
        cp.wait()
    for j in range(nchunk):
        pltpu.sync_copy(rows_v.at[j], q_hbm.at[base + j])
    acc_v[...] = jnp.zeros((16,), jnp.float32)

    def _row(r, j):
        a0 = rows_v[j, r, pl.ds(0, 16)] - x_v[j, r, pl.ds(0, 16)]
        a1 = rows_v[j, r, pl.ds(16, 16)] - x_v[j, r, pl.ds(16, 16)]
        acc_v[...] = acc_v[...] + a0 * a0 + a1 * a1
        return j

    for j in range(nchunk):
        lax.fori_loop(0, _CHUNK, _row, j)
    pltpu.sync_copy(acc_v, part_hbm.at[wid])


def kernel(z_e, codebook):
    B, H, W, D = z_e.shape
    d = D // N_PARTS
    n_tok = B * H * W
    n_blk = n_tok // _TM
    total = n_tok * N_PARTS
    nchunk = total // (_NW * _CHUNK)

    x_v = z_e.reshape(n_tok, D)

    idx3 = pl.pallas_call(
        _vq_block,
        grid=(n_blk,),
        in_specs=[
            pl.BlockSpec((_TM, D), lambda i: (i, 0)),
            pl.BlockSpec((N_EMBED, d), lambda i: (0, 0)),
        ],
        out_specs=pl.BlockSpec((1, _TM, N_PARTS), lambda i: (i, 0, 0)),
        out_shape=jax.ShapeDtypeStruct((n_blk, _TM, N_PARTS), jnp.int32),
        scratch_shapes=[
            pltpu.VMEM((d, N_EMBED), jnp.float32),
            pltpu.VMEM((1, N_EMBED), jnp.float32),
        ],
    )(x_v, codebook)

    idx_chunks = idx3.reshape(total // _CHUNK, _CHUNK)
    x_rows = z_e.reshape(total // _CHUNK, _CHUNK, d)

    sc = functools.partial(
        pl.kernel,
        mesh=plsc.VectorSubcoreMesh(core_axis_name="c", subcore_axis_name="s"),
        compiler_params=pltpu.CompilerParams(use_tc_tiling_on_sc=False),
        out_type=[
            jax.ShapeDtypeStruct((total // _CHUNK, _CHUNK, d), jnp.float32),
            jax.ShapeDtypeStruct((_NW, 16), jnp.float32),
        ],
        scratch_types=[
            pltpu.VMEM((nchunk, _CHUNK), jnp.int32),
            pltpu.VMEM((nchunk, _CHUNK, d), jnp.float32),
            pltpu.VMEM((nchunk, _CHUNK, d), jnp.float32),
            pltpu.VMEM((16,), jnp.float32),
            pltpu.SemaphoreType.DMA,
        ],
    )(_sc_gather_commit)
    q_rows, partials = sc(codebook, idx_chunks, x_rows)

    z_q = q_rows.reshape(B, H, W, D)
    indices = idx3.reshape(B, H * W, N_PARTS)
    commit_loss = (COMMITMENT * jnp.sum(partials)
                   / jnp.float32(n_tok * d))
    return z_q, indices, commit_loss


# speedup vs baseline: 3.2838x; 1.0279x over previous
import functools

import jax
import jax.numpy as jnp
from jax import lax
from jax.experimental import pallas as pl
from jax.experimental.pallas import tpu as pltpu
from jax.experimental.pallas import tpu_sc as plsc

N_PARTS = 2
N_EMBED = 8192
CODE_DIM = 32
COMMITMENT = 1.0

_TM = 512
_SLAB = _TM // 128
_NC, _NS = 2, 16
_NW = _NC * _NS
_CHUNK = 128


def _vq_block(x_ref, cb_ref, idx_ref, idxp_ref, cbt_ref, csq_ref):
    i = pl.program_id(0)

    @pl.when(i == 0)
    def _build_aug():
        cb0 = cb_ref[...]
        cbt_ref[...] = (-2.0 * cb0).T
        csq_ref[...] = jnp.sum(cb0 * cb0, axis=-1)[None, :]

    x64 = x_ref[...]
    csq = csq_ref[...]
    idx_parts = []
    for p in range(N_PARTS):
        x = x64[:, p * CODE_DIM:(p + 1) * CODE_DIM]
        scores = jnp.dot(x, cbt_ref[...],
                         preferred_element_type=jnp.float32) + csq
        idx = jnp.argmin(scores, axis=-1).astype(jnp.int32)
        idx_parts.append(idx[:, None])
        for r in range(_SLAB):
            idxp_ref[p, 0, r, :] = idx[r * 128:(r + 1) * 128]
    idx_ref[0, :, :] = jnp.concatenate(idx_parts, axis=1)


def _sc_gather_commit(n_tok, cb_hbm, idxp_hbm, x_hbm, q_hbm, part_hbm,
                      idx_v, rows_v, x_v, acc_v, sem):
    wid = lax.axis_index("s") * _NC + lax.axis_index("c")
    per_part = _NW // N_PARTS
    span = n_tok // per_part
    nchunk = span // _CHUNK
    p = wid // per_part
    a = (wid % per_part) * span
    rowbase = (p * n_tok + a) // _CHUNK
    pltpu.sync_copy(idxp_hbm.at[pl.ds(rowbase, nchunk)], idx_v)
    copies = [
        pltpu.async_copy(cb_hbm.at[idx_v.at[j]], rows_v.at[j], sem)
        for j in range(nchunk)
    ]
    pltpu.sync_copy(
        x_hbm.at[pl.ds(a, span), pl.ds(p * CODE_DIM, CODE_DIM)], x_v)
    for cp in copies:
        cp.wait()
    for j in range(nchunk):
        pltpu.sync_copy(
            rows_v.at[j],
            q_hbm.at[pl.ds(a + j * _CHUNK, _CHUNK),
                     pl.ds(p * CODE_DIM, CODE_DIM)])
    acc_v[...] = jnp.zeros((16,), jnp.float32)

    def _row(r, j):
        a0 = rows_v[j, r, pl.ds(0, 16)] - x_v[j * _CHUNK + r, pl.ds(0, 16)]
        a1 = rows_v[j, r, pl.ds(16, 16)] - x_v[j * _CHUNK + r, pl.ds(16, 16)]
        acc_v[...] = acc_v[...] + a0 * a0 + a1 * a1
        return j

    for j in range(nchunk):
        lax.fori_loop(0, _CHUNK, _row, j)
    pltpu.sync_copy(acc_v, part_hbm.at[wid])


def kernel(z_e, codebook):
    B, H, W, D = z_e.shape
    d = D // N_PARTS
    n_tok = B * H * W
    n_blk = n_tok // _TM
    span = n_tok // (_NW // N_PARTS)
    nchunk = span // _CHUNK

    x64 = z_e.reshape(n_tok, D)

    idx3, idxp = pl.pallas_call(
        _vq_block,
        grid=(n_blk,),
        in_specs=[
            pl.BlockSpec((_TM, D), lambda i: (i, 0)),
            pl.BlockSpec((N_EMBED, d), lambda i: (0, 0)),
        ],
        out_specs=[
            pl.BlockSpec((1, _TM, N_PARTS), lambda i: (i, 0, 0)),
            pl.BlockSpec((N_PARTS, 1, _SLAB, 128), lambda i: (0, i, 0, 0)),
        ],
        out_shape=[
            jax.ShapeDtypeStruct((n_blk, _TM, N_PARTS), jnp.int32),
            jax.ShapeDtypeStruct((N_PARTS, n_blk, _SLAB, 128), jnp.int32),
        ],
        scratch_shapes=[
            pltpu.VMEM((d, N_EMBED), jnp.float32),
            pltpu.VMEM((1, N_EMBED), jnp.float32),
        ],
    )(x64, codebook)

    idxp2 = idxp.reshape(N_PARTS * n_tok // _CHUNK, _CHUNK)

    sc = functools.partial(
        pl.kernel,
        mesh=plsc.VectorSubcoreMesh(core_axis_name="c", subcore_axis_name="s"),
        compiler_params=pltpu.CompilerParams(use_tc_tiling_on_sc=False),
        out_type=[
            jax.ShapeDtypeStruct((n_tok, D), jnp.float32),
            jax.ShapeDtypeStruct((_NW, 16), jnp.float32),
        ],
        scratch_types=[
            pltpu.VMEM((nchunk, _CHUNK), jnp.int32),
            pltpu.VMEM((nchunk, _CHUNK, d), jnp.float32),
            pltpu.VMEM((span, d), jnp.float32),
            pltpu.VMEM((16,), jnp.float32),
            pltpu.SemaphoreType.DMA,
        ],
    )(functools.partial(_sc_gather_commit, n_tok))
    q64, partials = sc(codebook, idxp2, x64)

    z_q = q64.reshape(B, H, W, D)
    indices = idx3.reshape(B, H * W, N_PARTS)
    commit_loss = (COMMITMENT * jnp.sum(partials)
                   / jnp.float32(n_tok * d))
    return z_q, indices, commit_loss


# speedup vs baseline: 3.3199x; 1.0110x over previous
import functools

import jax
import jax.numpy as jnp
from jax import lax
from jax.experimental import pallas as pl
from jax.experimental.pallas import tpu as pltpu
from jax.experimental.pallas import tpu_sc as plsc

N_PARTS = 2
N_EMBED = 8192
CODE_DIM = 32
COMMITMENT = 1.0

_TM = 512
_SLAB = _TM // 128
_NC, _NS = 2, 16
_NW = _NC * _NS
_CHUNK = 128


def _vq_block(x_ref, cb_ref, idxp_ref, cbt_ref, csq_ref):
    i = pl.program_id(0)

    @pl.when(i == 0)
    def _build_aug():
        cb0 = cb_ref[...]
        cbt_ref[...] = (-2.0 * cb0).T
        csq_ref[...] = jnp.sum(cb0 * cb0, axis=-1)[None, :]

    x64 = x_ref[...]
    csq = csq_ref[...]
    for p in range(N_PARTS):
        x = x64[:, p * CODE_DIM:(p + 1) * CODE_DIM]
        scores = jnp.dot(x, cbt_ref[...],
                         preferred_element_type=jnp.float32) + csq
        idx = jnp.argmin(scores, axis=-1).astype(jnp.int32)
        for r in range(_SLAB):
            idxp_ref[p, 0, r, :] = idx[r * 128:(r + 1) * 128]


def _sc_gather_commit(n_tok, cb_hbm, idxp_hbm, x_hbm, q_hbm, part_hbm,
                      idx_v, rows_v, x_v, acc_v, sem):
    wid = lax.axis_index("s") * _NC + lax.axis_index("c")
    per_part = _NW // N_PARTS
    span = n_tok // per_part
    nchunk = span // _CHUNK
    p = wid // per_part
    a = (wid % per_part) * span
    rowbase = (p * n_tok + a) // _CHUNK
    pltpu.sync_copy(idxp_hbm.at[pl.ds(rowbase, nchunk)], idx_v)
    copies = [
        pltpu.async_copy(cb_hbm.at[idx_v.at[j]], rows_v.at[j], sem)
        for j in range(nchunk)
    ]
    pltpu.sync_copy(
        x_hbm.at[pl.ds(a, span), pl.ds(p * CODE_DIM, CODE_DIM)], x_v)
    for cp in copies:
        cp.wait()
    for j in range(nchunk):
        pltpu.sync_copy(
            rows_v.at[j],
            q_hbm.at[pl.ds(a + j * _CHUNK, _CHUNK),
                     pl.ds(p * CODE_DIM, CODE_DIM)])
    acc_v[...] = jnp.zeros((16,), jnp.float32)

    def _row(r, j):
        a0 = rows_v[j, r, pl.ds(0, 16)] - x_v[j * _CHUNK + r, pl.ds(0, 16)]
        a1 = rows_v[j, r, pl.ds(16, 16)] - x_v[j * _CHUNK + r, pl.ds(16, 16)]
        acc_v[...] = acc_v[...] + a0 * a0 + a1 * a1
        return j

    for j in range(nchunk):
        lax.fori_loop(0, _CHUNK, _row, j)
    pltpu.sync_copy(acc_v, part_hbm.at[wid])


def kernel(z_e, codebook):
    B, H, W, D = z_e.shape
    d = D // N_PARTS
    n_tok = B * H * W
    n_blk = n_tok // _TM
    span = n_tok // (_NW // N_PARTS)
    nchunk = span // _CHUNK

    x64 = z_e.reshape(n_tok, D)

    idxp = pl.pallas_call(
        _vq_block,
        grid=(n_blk,),
        in_specs=[
            pl.BlockSpec((_TM, D), lambda i: (i, 0)),
            pl.BlockSpec((N_EMBED, d), lambda i: (0, 0)),
        ],
        out_specs=pl.BlockSpec((N_PARTS, 1, _SLAB, 128), lambda i: (0, i, 0, 0)),
        out_shape=jax.ShapeDtypeStruct((N_PARTS, n_blk, _SLAB, 128), jnp.int32),
        scratch_shapes=[
            pltpu.VMEM((d, N_EMBED), jnp.float32),
            pltpu.VMEM((1, N_EMBED), jnp.float32),
        ],
    )(x64, codebook)

    idxp2 = idxp.reshape(N_PARTS * n_tok // _CHUNK, _CHUNK)

    sc = functools.partial(
        pl.kernel,
        mesh=plsc.VectorSubcoreMesh(core_axis_name="c", subcore_axis_name="s"),
        compiler_params=pltpu.CompilerParams(use_tc_tiling_on_sc=False),
        out_type=[
            jax.ShapeDtypeStruct((n_tok, D), jnp.float32),
            jax.ShapeDtypeStruct((_NW, 16), jnp.float32),
        ],
        scratch_types=[
            pltpu.VMEM((nchunk, _CHUNK), jnp.int32),
            pltpu.VMEM((nchunk, _CHUNK, d), jnp.float32),
            pltpu.VMEM((span, d), jnp.float32),
            pltpu.VMEM((16,), jnp.float32),
            pltpu.SemaphoreType.DMA,
        ],
    )(functools.partial(_sc_gather_commit, n_tok))
    q64, partials = sc(codebook, idxp2, x64)

    z_q = q64.reshape(B, H, W, D)
    indices = (idxp.reshape(N_PARTS, n_tok).transpose(1, 0)
               .reshape(B, H * W, N_PARTS))
    commit_loss = (COMMITMENT * jnp.sum(partials)
                   / jnp.float32(n_tok * d))
    return z_q, indices, commit_loss


# speedup vs baseline: 3.3277x; 1.0024x over previous
import functools

import jax
import jax.numpy as jnp
from jax import lax
from jax.experimental import pallas as pl
from jax.experimental.pallas import tpu as pltpu
from jax.experimental.pallas import tpu_sc as plsc

N_PARTS = 2
N_EMBED = 8192
CODE_DIM = 32
COMMITMENT = 1.0

_TM = 512
_SLAB = _TM // 128
_NC, _NS = 2, 16
_NW = _NC * _NS
_CHUNK = 128


def _vq_block(x_ref, cb_ref, idxp_ref, cbt_ref, csq_ref):
    i = pl.program_id(0)

    @pl.when(i == 0)
    def _build_aug():
        cb0 = cb_ref[...]
        cbt_ref[...] = (-2.0 * cb0).T
        csq_ref[...] = jnp.sum(cb0 * cb0, axis=-1)[None, :]

    x64 = x_ref[...]
    csq = csq_ref[...]
    for p in range(N_PARTS):
        x = x64[:, p * CODE_DIM:(p + 1) * CODE_DIM]
        scores = jnp.dot(x, cbt_ref[...],
                         preferred_element_type=jnp.float32) + csq
        idx = jnp.argmin(scores, axis=-1).astype(jnp.int32)
        for r in range(_SLAB):
            idxp_ref[p, 0, r, :] = idx[r * 128:(r + 1) * 128]


def _sc_gather_commit(n_tok, cb_hbm, idxp_hbm, x_hbm, q_hbm, part_hbm,
                      idx_v, rows_v, x_v, acc_v, cb_sh, sem):
    sid = lax.axis_index("s")
    wid = sid * _NC + lax.axis_index("c")
    per_part = _NW // N_PARTS
    span = n_tok // per_part
    nchunk = span // _CHUNK
    p = wid // per_part
    a = (wid % per_part) * span
    rowbase = (p * n_tok + a) // _CHUNK

    @pl.when(sid == 0)
    def _stage():
        pltpu.sync_copy(cb_hbm, cb_sh)

    pltpu.sync_copy(idxp_hbm.at[pl.ds(rowbase, nchunk)], idx_v)
    plsc.subcore_barrier()
    copies = [
        pltpu.async_copy(cb_sh.at[idx_v.at[j]], rows_v.at[j], sem)
        for j in range(nchunk)
    ]
    pltpu.sync_copy(
        x_hbm.at[pl.ds(a, span), pl.ds(p * CODE_DIM, CODE_DIM)], x_v)
    for cp in copies:
        cp.wait()
    for j in range(nchunk):
        pltpu.sync_copy(
            rows_v.at[j],
            q_hbm.at[pl.ds(a + j * _CHUNK, _CHUNK),
                     pl.ds(p * CODE_DIM, CODE_DIM)])
    acc_v[...] = jnp.zeros((16,), jnp.float32)

    def _row(r, j):
        a0 = rows_v[j, r, pl.ds(0, 16)] - x_v[j * _CHUNK + r, pl.ds(0, 16)]
        a1 = rows_v[j, r, pl.ds(16, 16)] - x_v[j * _CHUNK + r, pl.ds(16, 16)]
        acc_v[...] = acc_v[...] + a0 * a0 + a1 * a1
        return j

    for j in range(nchunk):
        lax.fori_loop(0, _CHUNK, _row, j)
    pltpu.sync_copy(acc_v, part_hbm.at[wid])


def kernel(z_e, codebook):
    B, H, W, D = z_e.shape
    d = D // N_PARTS
    n_tok = B * H * W
    n_blk = n_tok // _TM
    span = n_tok // (_NW // N_PARTS)
    nchunk = span // _CHUNK

    x64 = z_e.reshape(n_tok, D)

    idxp = pl.pallas_call(
        _vq_block,
        grid=(n_blk,),
        in_specs=[
            pl.BlockSpec((_TM, D), lambda i: (i, 0)),
            pl.BlockSpec((N_EMBED, d), lambda i: (0, 0)),
        ],
        out_specs=pl.BlockSpec((N_PARTS, 1, _SLAB, 128), lambda i: (0, i, 0, 0)),
        out_shape=jax.ShapeDtypeStruct((N_PARTS, n_blk, _SLAB, 128), jnp.int32),
        scratch_shapes=[
            pltpu.VMEM((d, N_EMBED), jnp.float32),
            pltpu.VMEM((1, N_EMBED), jnp.float32),
        ],
    )(x64, codebook)

    idxp2 = idxp.reshape(N_PARTS * n_tok // _CHUNK, _CHUNK)

    sc = functools.partial(
        pl.kernel,
        mesh=plsc.VectorSubcoreMesh(core_axis_name="c", subcore_axis_name="s"),
        compiler_params=pltpu.CompilerParams(use_tc_tiling_on_sc=False),
        out_type=[
            jax.ShapeDtypeStruct((n_tok, D), jnp.float32),
            jax.ShapeDtypeStruct((_NW, 16), jnp.float32),
        ],
        scratch_types=[
            pltpu.VMEM((nchunk, _CHUNK), jnp.int32),
            pltpu.VMEM((nchunk, _CHUNK, d), jnp.float32),
            pltpu.VMEM((span, d), jnp.float32),
            pltpu.VMEM((16,), jnp.float32),
            pltpu.VMEM_SHARED((N_EMBED, d), jnp.float32),
            pltpu.SemaphoreType.DMA,
        ],
    )(functools.partial(_sc_gather_commit, n_tok))
    q64, partials = sc(codebook, idxp2, x64)

    z_q = q64.reshape(B, H, W, D)
    indices = (idxp.reshape(N_PARTS, n_tok).transpose(1, 0)
               .reshape(B, H * W, N_PARTS))
    commit_loss = (COMMITMENT * jnp.sum(partials)
                   / jnp.float32(n_tok * d))
    return z_q, indices, commit_loss
